# Initial kernel scaffold; baseline (speedup 1.0000x reference)
import jax
import jax.numpy as jnp
from jax import lax
from jax.experimental import pallas as pl
from jax.experimental.pallas import tpu as pltpu
from jax.experimental.pallas import tpu_sc as plsc

VOCAB = 100000
HIDDEN = 128
MAX_POS = 512
BATCH = 1024
EPS = 1e-12

LANES = 16
HREG = HIDDEN // LANES
CHUNK = 128
NCHUNK = MAX_POS // CHUNK


def _rsqrt(a):
    i = lax.bitcast_convert_type(a, jnp.int32)
    i = jnp.int32(0x5F3759DF) - lax.shift_right_arithmetic(i, 1)
    y = lax.bitcast_convert_type(i, jnp.float32)
    for _ in range(3):
        y = y * (1.5 - 0.5 * a * y * y)
    return y


def _sc_body(ids_hbm, tt_hbm, word_hbm, pos_hbm, type_hbm, gamma_hbm,
             beta_hbm, out_hbm, pos_v, type_v, gb_v, idx_v, tt_v, rows_v,
             sem):
    info = plsc.get_sparse_core_info()
    nc = info.num_cores
    wid = lax.axis_index("s") * nc + lax.axis_index("c")
    nw = nc * info.num_subcores
    rows_per_w = BATCH // nw

    pltpu.sync_copy(pos_hbm, pos_v)
    pltpu.sync_copy(type_hbm, type_v)
    pltpu.sync_copy(gamma_hbm, gb_v.at[0])
    pltpu.sync_copy(beta_hbm, gb_v.at[1])

    ty0 = [type_v[0, pl.ds(h * LANES, LANES)] for h in range(HREG)]
    tyd = [type_v[1, pl.ds(h * LANES, LANES)] - ty0[h] for h in range(HREG)]
    gam = [gb_v[0, pl.ds(h * LANES, LANES)] for h in range(HREG)]
    bet = [gb_v[1, pl.ds(h * LANES, LANES)] for h in range(HREG)]

    def chunk_body(step, _):
        b = wid * rows_per_w + step // NCHUNK
        base = (step % NCHUNK) * CHUNK
        pltpu.sync_copy(ids_hbm.at[b, pl.ds(base, CHUNK)], idx_v)
        pltpu.sync_copy(tt_hbm.at[b, pl.ds(base, CHUNK)], tt_v)
        pltpu.async_copy(word_hbm.at[idx_v], rows_v, sem).wait()

        def token_body(t, _):
            tf = tt_v[t].astype(jnp.float32)
            p = base + t
            acc = jnp.zeros((LANES,), jnp.float32)
            acc2 = jnp.zeros((LANES,), jnp.float32)
            xs = []
            for h in range(HREG):
                x = (rows_v[t, pl.ds(h * LANES, LANES)]
                     + pos_v[p, pl.ds(h * LANES, LANES)]
                     + (ty0[h] + tf * tyd[h]))
                acc = acc + x
                acc2 = acc2 + x * x
                xs.append(x)
            mean = jnp.sum(acc) * (1.0 / HIDDEN)
            var = jnp.sum(acc2) * (1.0 / HIDDEN) - mean * mean
            rstd = _rsqrt(jnp.maximum(var, 0.0) + EPS)
            for h in range(HREG):
                out = (xs[h] - mean) * rstd * gam[h] + bet[h]
                rows_v[t, pl.ds(h * LANES, LANES)] = out
            return 0

        lax.fori_loop(0, CHUNK, token_body, 0)
        pltpu.sync_copy(rows_v, out_hbm.at[b, pl.ds(base, CHUNK)])
        return 0

    lax.fori_loop(0, rows_per_w * NCHUNK, chunk_body, 0)


def kernel(input_ids, token_type_ids, word_table, pos_table, type_table,
           ln_gamma, ln_beta):
    mesh = plsc.VectorSubcoreMesh(core_axis_name="c", subcore_axis_name="s")
    f = pl.kernel(
        _sc_body,
        out_type=jax.ShapeDtypeStruct((BATCH, MAX_POS, HIDDEN), jnp.float32),
        mesh=mesh,
        scratch_types=[
            pltpu.VMEM((MAX_POS, HIDDEN), jnp.float32),
            pltpu.VMEM((2, HIDDEN), jnp.float32),
            pltpu.VMEM((2, HIDDEN), jnp.float32),
            pltpu.VMEM((CHUNK,), jnp.int32),
            pltpu.VMEM((CHUNK,), jnp.int32),
            pltpu.VMEM((CHUNK, HIDDEN), jnp.float32),
            pltpu.SemaphoreType.DMA,
        ],
    )
    return f(input_ids.astype(jnp.int32), token_type_ids.astype(jnp.int32),
             word_table, pos_table, type_table, ln_gamma, ln_beta)


# speedup vs baseline: 2.8008x; 2.8008x over previous
import jax
import jax.numpy as jnp
from jax import lax
from jax.experimental import pallas as pl
from jax.experimental.pallas import tpu as pltpu
from jax.experimental.pallas import tpu_sc as plsc

VOCAB = 100000
HIDDEN = 128
MAX_POS = 512
BATCH = 1024
EPS = 1e-12

LANES = 16
NCORES = 2
NSUBCORES = 16
HREG = HIDDEN // LANES
CHUNK = 128
NCHUNK = MAX_POS // CHUNK


def _rsqrt(a):
    i = lax.bitcast_convert_type(a, jnp.int32)
    i = jnp.int32(0x5F3759DF) - lax.shift_right_arithmetic(i, 1)
    y = lax.bitcast_convert_type(i, jnp.float32)
    for _ in range(3):
        y = y * (1.5 - 0.5 * a * y * y)
    return y


def _sc_body(ids_hbm, tt_hbm, word_hbm, pos_hbm, type_hbm, gamma_hbm,
             beta_hbm, out_hbm, pos_v, type_v, gb_v, idx_v, tt_v, rows_v,
             sem):
    wid = lax.axis_index("s") * NCORES + lax.axis_index("c")
    rows_per_w = BATCH // (NCORES * NSUBCORES)

    pltpu.sync_copy(pos_hbm, pos_v)
    pltpu.sync_copy(type_hbm, type_v)
    pltpu.sync_copy(gamma_hbm, gb_v.at[0])
    pltpu.sync_copy(beta_hbm, gb_v.at[1])

    ty0 = [type_v[0, pl.ds(h * LANES, LANES)] for h in range(HREG)]
    tyd = [type_v[1, pl.ds(h * LANES, LANES)] - ty0[h] for h in range(HREG)]
    gam = [gb_v[0, pl.ds(h * LANES, LANES)] for h in range(HREG)]
    bet = [gb_v[1, pl.ds(h * LANES, LANES)] for h in range(HREG)]

    def chunk_body(step, _):
        b = wid * rows_per_w + step // NCHUNK
        base = (step % NCHUNK) * CHUNK
        pltpu.sync_copy(ids_hbm.at[b, pl.ds(base, CHUNK)], idx_v)
        pltpu.sync_copy(tt_hbm.at[b, pl.ds(base, CHUNK)], tt_v)
        pltpu.async_copy(word_hbm.at[idx_v], rows_v, sem).wait()

        def group_body(g, _):
            t0 = g * LANES
            ttf = tt_v[pl.ds(t0, LANES)].astype(jnp.float32)
            for j in range(LANES):
                t = t0 + j
                tf = ttf[j]
                p = base + t
                acc = jnp.zeros((LANES,), jnp.float32)
                acc2 = jnp.zeros((LANES,), jnp.float32)
                xs = []
                for h in range(HREG):
                    x = (rows_v[t, pl.ds(h * LANES, LANES)]
                         + pos_v[p, pl.ds(h * LANES, LANES)]
                         + (ty0[h] + tf * tyd[h]))
                    acc = acc + x
                    acc2 = acc2 + x * x
                    xs.append(x)
                mean = jnp.sum(acc) * (1.0 / HIDDEN)
                var = jnp.sum(acc2) * (1.0 / HIDDEN) - mean * mean
                rstd = _rsqrt(jnp.maximum(var, 0.0) + EPS)
                for h in range(HREG):
                    out = (xs[h] - mean) * rstd * gam[h] + bet[h]
                    rows_v[t, pl.ds(h * LANES, LANES)] = out
            return 0

        lax.fori_loop(0, CHUNK // LANES, group_body, 0)
        pltpu.sync_copy(rows_v, out_hbm.at[b, pl.ds(base, CHUNK)])
        return 0

    lax.fori_loop(0, rows_per_w * NCHUNK, chunk_body, 0)


def kernel(input_ids, token_type_ids, word_table, pos_table, type_table,
           ln_gamma, ln_beta):
    mesh = plsc.VectorSubcoreMesh(core_axis_name="c", subcore_axis_name="s")
    f = pl.kernel(
        _sc_body,
        out_type=jax.ShapeDtypeStruct((BATCH, MAX_POS, HIDDEN), jnp.float32),
        mesh=mesh,
        compiler_params=pltpu.CompilerParams(needs_layout_passes=False),
        scratch_types=[
            pltpu.VMEM((MAX_POS, HIDDEN), jnp.float32),
            pltpu.VMEM((2, HIDDEN), jnp.float32),
            pltpu.VMEM((2, HIDDEN), jnp.float32),
            pltpu.VMEM((CHUNK,), jnp.int32),
            pltpu.VMEM((CHUNK,), jnp.int32),
            pltpu.VMEM((CHUNK, HIDDEN), jnp.float32),
            pltpu.SemaphoreType.DMA,
        ],
    )
    return f(input_ids.astype(jnp.int32), token_type_ids.astype(jnp.int32),
             word_table, pos_table, type_table, ln_gamma, ln_beta)


# speedup vs baseline: 3.1210x; 1.1143x over previous
import jax
import jax.numpy as jnp
from jax import lax
from jax.experimental import pallas as pl
from jax.experimental.pallas import tpu as pltpu
from jax.experimental.pallas import tpu_sc as plsc

VOCAB = 100000
HIDDEN = 128
MAX_POS = 512
BATCH = 1024
EPS = 1e-12

LANES = 16
NCORES = 2
NSUBCORES = 16
HREG = HIDDEN // LANES
CHUNK = 128
NCHUNK = MAX_POS // CHUNK


def _rsqrt(a):
    i = lax.bitcast_convert_type(a, jnp.int32)
    i = jnp.int32(0x5F3759DF) - lax.shift_right_arithmetic(i, 1)
    y = lax.bitcast_convert_type(i, jnp.float32)
    for _ in range(3):
        y = y * (1.5 - 0.5 * a * y * y)
    return y


def _sc_body(ids_hbm, tt_hbm, word_hbm, pos_hbm, type_hbm, gamma_hbm,
             beta_hbm, out_hbm, pos_v, type_v, gb_v, idx_v, tt_v, rows_v,
             gsem0, gsem1, wsem0, wsem1):
    gsem = (gsem0, gsem1)
    wsem = (wsem0, wsem1)
    wid = lax.axis_index("s") * NCORES + lax.axis_index("c")
    rows_per_w = BATCH // (NCORES * NSUBCORES)
    nsteps = rows_per_w * NCHUNK

    pltpu.sync_copy(pos_hbm, pos_v)
    pltpu.sync_copy(type_hbm, type_v)
    pltpu.sync_copy(gamma_hbm, gb_v.at[0])
    pltpu.sync_copy(beta_hbm, gb_v.at[1])

    ty0 = [type_v[0, pl.ds(h * LANES, LANES)] for h in range(HREG)]
    tyd = [type_v[1, pl.ds(h * LANES, LANES)] - ty0[h] for h in range(HREG)]
    gam = [gb_v[0, pl.ds(h * LANES, LANES)] for h in range(HREG)]
    bet = [gb_v[1, pl.ds(h * LANES, LANES)] for h in range(HREG)]

    def slices(s):
        b = wid * rows_per_w + s // NCHUNK
        base = (s % NCHUNK) * CHUNK
        return b, base

    def issue_gather(s, buf):
        b, base = slices(s)
        pltpu.sync_copy(ids_hbm.at[b, pl.ds(base, CHUNK)], idx_v.at[buf])
        pltpu.sync_copy(tt_hbm.at[b, pl.ds(base, CHUNK)], tt_v.at[buf])
        pltpu.async_copy(word_hbm.at[idx_v.at[buf]], rows_v.at[buf],
                         gsem[buf])

    def wait_gather(buf):
        pltpu.make_async_copy(word_hbm.at[idx_v.at[buf]], rows_v.at[buf],
                              gsem[buf]).wait()

    def issue_wb(s, buf):
        b, base = slices(s)
        pltpu.async_copy(rows_v.at[buf], out_hbm.at[b, pl.ds(base, CHUNK)],
                         wsem[buf])

    def wait_wb(buf):
        pltpu.make_async_copy(rows_v.at[buf],
                              out_hbm.at[0, pl.ds(0, CHUNK)],
                              wsem[buf]).wait()

    def compute_step(s, buf):
        _, base = slices(s)

        def group_body(g, _):
            t0 = g * LANES
            ttf = tt_v[buf, pl.ds(t0, LANES)].astype(jnp.float32)
            for j in range(LANES):
                t = t0 + j
                tf = ttf[j]
                p = base + t
                acc = jnp.zeros((LANES,), jnp.float32)
                acc2 = jnp.zeros((LANES,), jnp.float32)
                xs = []
                for h in range(HREG):
                    x = (rows_v[buf, t, pl.ds(h * LANES, LANES)]
                         + pos_v[p, pl.ds(h * LANES, LANES)]
                         + (ty0[h] + tf * tyd[h]))
                    acc = acc + x
                    acc2 = acc2 + x * x
                    xs.append(x)
                mean = jnp.sum(acc) * (1.0 / HIDDEN)
                var = jnp.sum(acc2) * (1.0 / HIDDEN) - mean * mean
                rstd = _rsqrt(jnp.maximum(var, 0.0) + EPS)
                for h in range(HREG):
                    out = (xs[h] - mean) * rstd * gam[h] + bet[h]
                    rows_v[buf, t, pl.ds(h * LANES, LANES)] = out
            return 0

        lax.fori_loop(0, CHUNK // LANES, group_body, 0)

    issue_gather(0, 0)

    def outer(it, _):
        for buf in range(2):
            s = it * 2 + buf
            nxt = s + 1

            @pl.when(nxt < nsteps)
            def _():
                @pl.when(s >= 1)
                def _():
                    wait_wb(1 - buf)
                issue_gather(nxt, 1 - buf)

            wait_gather(buf)
            compute_step(s, buf)
            issue_wb(s, buf)
        return 0

    lax.fori_loop(0, nsteps // 2, outer, 0)
    wait_wb(0)
    wait_wb(1)


def kernel(input_ids, token_type_ids, word_table, pos_table, type_table,
           ln_gamma, ln_beta):
    mesh = plsc.VectorSubcoreMesh(core_axis_name="c", subcore_axis_name="s")
    f = pl.kernel(
        _sc_body,
        out_type=jax.ShapeDtypeStruct((BATCH, MAX_POS, HIDDEN), jnp.float32),
        mesh=mesh,
        compiler_params=pltpu.CompilerParams(needs_layout_passes=False),
        scratch_types=[
            pltpu.VMEM((MAX_POS, HIDDEN), jnp.float32),
            pltpu.VMEM((2, HIDDEN), jnp.float32),
            pltpu.VMEM((2, HIDDEN), jnp.float32),
            pltpu.VMEM((2, CHUNK), jnp.int32),
            pltpu.VMEM((2, CHUNK), jnp.int32),
            pltpu.VMEM((2, CHUNK, HIDDEN), jnp.float32),
            pltpu.SemaphoreType.DMA,
            pltpu.SemaphoreType.DMA,
            pltpu.SemaphoreType.DMA,
            pltpu.SemaphoreType.DMA,
        ],
    )
    return f(input_ids.astype(jnp.int32), token_type_ids.astype(jnp.int32),
             word_table, pos_table, type_table, ln_gamma, ln_beta)
